# Initial kernel scaffold; baseline (speedup 1.0000x reference)
import jax
import jax.numpy as jnp
from jax.experimental import pallas as pl


def kernel(temporal_features, edge_index, W1, b1, W2, b2, W3, b3, conv_w, conv_b):
    raise NotImplementedError("write your pallas kernel here")


# speedup vs baseline: 23.5164x; 23.5164x over previous
import jax
import jax.numpy as jnp
from jax import lax
from jax.experimental import pallas as pl
from jax.experimental.pallas import tpu as pltpu
from jax.experimental.pallas import tpu_sc as plsc

N = 10000
E = 160000
T = 12
F = 32
D = T * F
DQ = D // 4
TQ = T // 4

NS = 16
CH = 80
LCH = 128
EP = NS * CH * LCH
NPAD = 10496
SLAB = NPAD // NS
BN = 400
BNP = 656
GRID_N = N // BN
GRID_P = NPAD // BNP

_sc_mesh = plsc.VectorSubcoreMesh(core_axis_name="c", subcore_axis_name="s")
_sc_params = pltpu.CompilerParams(use_tc_tiling_on_sc=False)


def _deg_body(srcI, dstI, ones_h, z8, degS, degD, idx_v, ones_v, deg_sh):
    c = lax.axis_index("c")
    s = lax.axis_index("s")
    sl = pl.ds(s * SLAB, SLAB)
    pltpu.sync_copy(z8, deg_sh.at[sl])
    pltpu.sync_copy(ones_h, ones_v)

    @pl.when(c == 0)
    def _():
        pltpu.sync_copy(srcI.at[s], idx_v)

    @pl.when(c == 1)
    def _():
        pltpu.sync_copy(dstI.at[s], idx_v)

    plsc.subcore_barrier()

    def body(j, carry):
        pltpu.sync_copy(ones_v, deg_sh.at[idx_v.at[j]], add=True)
        return carry

    lax.fori_loop(0, CH, body, 0)
    plsc.subcore_barrier()

    @pl.when(c == 0)
    def _():
        pltpu.sync_copy(deg_sh.at[sl], degS.at[sl])

    @pl.when(c == 1)
    def _():
        pltpu.sync_copy(deg_sh.at[sl], degD.at[sl])


_deg_call = pl.kernel(
    _deg_body,
    out_type=(
        jax.ShapeDtypeStruct((NPAD, 8), jnp.float32),
        jax.ShapeDtypeStruct((NPAD, 8), jnp.float32),
    ),
    mesh=_sc_mesh,
    scratch_types=[
        pltpu.VMEM((CH, LCH), jnp.int32),
        pltpu.VMEM((LCH, 8), jnp.float32),
        pltpu.VMEM_SHARED((NPAD, 8), jnp.float32),
    ],
    compiler_params=_sc_params,
)


def _agg_body(t0, t1, srcI, dstI, zrows, agg0, agg1, src_v, dst_v, rows, agg_sh, sem):
    c = lax.axis_index("c")
    s = lax.axis_index("s")
    sl = pl.ds(s * SLAB, SLAB)
    pltpu.sync_copy(zrows, agg_sh.at[sl])
    pltpu.sync_copy(srcI.at[s], src_v)
    pltpu.sync_copy(dstI.at[s], dst_v)
    plsc.subcore_barrier()

    def run(tab):
        def body(j, carry):
            pltpu.async_copy(tab.at[src_v.at[j]], rows, sem).wait()
            pltpu.sync_copy(rows, agg_sh.at[dst_v.at[j]], add=True)
            return carry

        lax.fori_loop(0, CH, body, 0)

    @pl.when(c == 0)
    def _():
        run(t0)

    @pl.when(c == 1)
    def _():
        run(t1)

    plsc.subcore_barrier()

    @pl.when(c == 0)
    def _():
        pltpu.sync_copy(agg_sh.at[sl], agg0.at[sl])

    @pl.when(c == 1)
    def _():
        pltpu.sync_copy(agg_sh.at[sl], agg1.at[sl])


_agg_call = pl.kernel(
    _agg_body,
    out_type=(
        jax.ShapeDtypeStruct((NPAD, DQ), jnp.float32),
        jax.ShapeDtypeStruct((NPAD, DQ), jnp.float32),
    ),
    mesh=_sc_mesh,
    scratch_types=[
        pltpu.VMEM((CH, LCH), jnp.int32),
        pltpu.VMEM((CH, LCH), jnp.int32),
        pltpu.VMEM((LCH, DQ), jnp.float32),
        pltpu.VMEM_SHARED((NPAD, DQ), jnp.float32),
        pltpu.SemaphoreType.DMA,
    ],
    compiler_params=_sc_params,
)


def _store_quarters(hw, orefs):
    for q, oref in enumerate(orefs):
        oref[...] = hw[:, q * TQ:(q + 1) * TQ]


def _first_body(x_ref, dS_ref, w_ref, o0_ref, o1_ref, o2_ref, o3_ref):
    b = x_ref.shape[0]
    sn = lax.rsqrt(jnp.maximum(dS_ref[:, :1], 1.0)).reshape(b, 1, 1)
    hw = jnp.dot(
        x_ref[...].reshape(b * T, F), w_ref[...],
        preferred_element_type=jnp.float32,
    ).reshape(b, T, F) * sn
    _store_quarters(hw, (o0_ref, o1_ref, o2_ref, o3_ref))


def _mid_body(a0_ref, a1_ref, a2_ref, a3_ref, dD_ref, dS_ref, w_ref, b_ref,
              o0_ref, o1_ref, o2_ref, o3_ref):
    b = a0_ref.shape[0]
    agg = jnp.concatenate(
        [a0_ref[...], a1_ref[...], a2_ref[...], a3_ref[...]], axis=1)
    dn = lax.rsqrt(jnp.maximum(dD_ref[:, :1], 1.0)).reshape(b, 1, 1)
    h = jnp.maximum(agg * dn + b_ref[...].reshape(1, 1, F), 0.0)
    sn = lax.rsqrt(jnp.maximum(dS_ref[:, :1], 1.0)).reshape(b, 1, 1)
    hw = jnp.dot(
        h.reshape(b * T, F), w_ref[...],
        preferred_element_type=jnp.float32,
    ).reshape(b, T, F) * sn
    _store_quarters(hw, (o0_ref, o1_ref, o2_ref, o3_ref))


def _last_body(a0_ref, a1_ref, a2_ref, a3_ref, dD_ref, b_ref,
               wc0_ref, wc1_ref, wc2_ref, cb_ref, o_ref):
    b = a0_ref.shape[0]
    agg = jnp.concatenate(
        [a0_ref[...], a1_ref[...], a2_ref[...], a3_ref[...]], axis=1)
    dn = lax.rsqrt(jnp.maximum(dD_ref[:, :1], 1.0)).reshape(b, 1, 1)
    h = agg * dn + b_ref[...].reshape(1, 1, F)
    z = jnp.zeros((b, 1, F), jnp.float32)
    hp = jnp.concatenate([z, h, z], axis=1).reshape(b * (T + 2), F)

    def zmat(wc_ref):
        return jnp.dot(hp, wc_ref[...],
                       preferred_element_type=jnp.float32).reshape(b, T + 2, F)

    y = (zmat(wc0_ref)[:, 0:T] + zmat(wc1_ref)[:, 1:T + 1]
         + zmat(wc2_ref)[:, 2:T + 2]) + cb_ref[...].reshape(1, 1, F)
    o_ref[...] = y


def _node_spec(bn, t_len):
    return pl.BlockSpec((bn, t_len, F), lambda i: (i, 0, 0))


def _deg_spec(bn):
    return pl.BlockSpec((bn, 8), lambda i: (i, 0))


_w_spec = pl.BlockSpec((F, F), lambda i: (0, 0))
_b_spec = pl.BlockSpec((1, F), lambda i: (0, 0))


def _quarter_shapes(n):
    return tuple(jax.ShapeDtypeStruct((n, TQ, F), jnp.float32)
                 for _ in range(4))


def _first_call(x, degS, W):
    return pl.pallas_call(
        _first_body,
        grid=(GRID_N,),
        in_specs=[_node_spec(BN, T), _deg_spec(BN), _w_spec],
        out_specs=tuple(_node_spec(BN, TQ) for _ in range(4)),
        out_shape=_quarter_shapes(N),
    )(x, degS, W)


def _mid_call(a, degD, degS, W, bvec):
    return pl.pallas_call(
        _mid_body,
        grid=(GRID_P,),
        in_specs=[_node_spec(BNP, TQ)] * 4 + [_deg_spec(BNP), _deg_spec(BNP),
                                              _w_spec, _b_spec],
        out_specs=tuple(_node_spec(BNP, TQ) for _ in range(4)),
        out_shape=_quarter_shapes(NPAD),
    )(*a, degD, degS, W, bvec)


def _last_call(a, degD, bvec, wc0, wc1, wc2, cb):
    return pl.pallas_call(
        _last_body,
        grid=(GRID_P,),
        in_specs=[_node_spec(BNP, TQ)] * 4 + [_deg_spec(BNP), _b_spec,
                                              _w_spec, _w_spec, _w_spec,
                                              _b_spec],
        out_specs=_node_spec(BNP, T),
        out_shape=jax.ShapeDtypeStruct((NPAD, T, F), jnp.float32),
    )(*a, degD, bvec, wc0, wc1, wc2, cb)


@jax.jit
def kernel(temporal_features, edge_index, W1, b1, W2, b2, W3, b3, conv_w, conv_b):
    x = jnp.transpose(temporal_features, (0, 2, 1))
    src = edge_index[0]
    dst = edge_index[1]
    pad = EP - E
    srcA = jnp.concatenate([src, jnp.zeros((pad,), jnp.int32)]).reshape(NS, CH, LCH)
    srcD = jnp.concatenate([src, jnp.full((pad,), N, jnp.int32)]).reshape(NS, CH, LCH)
    dstI = jnp.concatenate([dst, jnp.full((pad,), N, jnp.int32)]).reshape(NS, CH, LCH)

    ones8 = jnp.ones((LCH, 8), jnp.float32)
    z8 = jnp.zeros((SLAB, 8), jnp.float32)
    zrows = jnp.zeros((SLAB, DQ), jnp.float32)

    degS, degD = _deg_call(srcD, dstI, ones8, z8)

    def agg(ts):
        a01 = _agg_call(ts[0].reshape(-1, DQ)[:N], ts[1].reshape(-1, DQ)[:N],
                        srcA, dstI, zrows)
        a23 = _agg_call(ts[2].reshape(-1, DQ)[:N], ts[3].reshape(-1, DQ)[:N],
                        srcA, dstI, zrows)
        return tuple(q.reshape(NPAD, TQ, F) for q in a01 + a23)

    t = _first_call(x, degS[:N], W1)
    a = agg(t)
    t = _mid_call(a, degD, degS, W2, b1.reshape(1, F))
    a = agg(t)
    t = _mid_call(a, degD, degS, W3, b2.reshape(1, F))
    a = agg(t)

    wc0 = conv_w[:, :, 0].T
    wc1 = conv_w[:, :, 1].T
    wc2 = conv_w[:, :, 2].T
    y = _last_call(a, degD, b3.reshape(1, F), wc0, wc1, wc2,
                   conv_b.reshape(1, F))
    return jnp.transpose(y[:N], (0, 2, 1))


# speedup vs baseline: 25.2841x; 1.0752x over previous
import jax
import jax.numpy as jnp
from jax import lax
from jax.experimental import pallas as pl
from jax.experimental.pallas import tpu as pltpu
from jax.experimental.pallas import tpu_sc as plsc

N = 10000
E = 160000
T = 12
F = 32
D = T * F
DQ = D // 4
TQ = T // 4

NS = 16
CH = 80
LCH = 128
EP = NS * CH * LCH
NPAD = 10496
SLAB = NPAD // NS
BN = 400
BNP = 656
GRID_N = N // BN
GRID_P = NPAD // BNP

_sc_mesh = plsc.VectorSubcoreMesh(core_axis_name="c", subcore_axis_name="s")
_sc_params = pltpu.CompilerParams(use_tc_tiling_on_sc=False)


def _deg_body(srcI, dstI, ones_h, z8, degS, degD, idx_v, ones_v, deg_sh):
    c = lax.axis_index("c")
    s = lax.axis_index("s")
    sl = pl.ds(s * SLAB, SLAB)
    pltpu.sync_copy(z8, deg_sh.at[sl])
    pltpu.sync_copy(ones_h, ones_v)

    @pl.when(c == 0)
    def _():
        pltpu.sync_copy(srcI.at[s], idx_v)

    @pl.when(c == 1)
    def _():
        pltpu.sync_copy(dstI.at[s], idx_v)

    plsc.subcore_barrier()

    def body(j, carry):
        pltpu.sync_copy(ones_v, deg_sh.at[idx_v.at[j]], add=True)
        return carry

    lax.fori_loop(0, CH, body, 0)
    plsc.subcore_barrier()

    @pl.when(c == 0)
    def _():
        pltpu.sync_copy(deg_sh.at[sl], degS.at[sl])

    @pl.when(c == 1)
    def _():
        pltpu.sync_copy(deg_sh.at[sl], degD.at[sl])


_deg_call = pl.kernel(
    _deg_body,
    out_type=(
        jax.ShapeDtypeStruct((NPAD, 8), jnp.float32),
        jax.ShapeDtypeStruct((NPAD, 8), jnp.float32),
    ),
    mesh=_sc_mesh,
    scratch_types=[
        pltpu.VMEM((CH, LCH), jnp.int32),
        pltpu.VMEM((LCH, 8), jnp.float32),
        pltpu.VMEM_SHARED((NPAD, 8), jnp.float32),
    ],
    compiler_params=_sc_params,
)


def _agg_body(t0, t1, t2, t3, srcI, dstI, zrows, a0, a1, a2, a3,
              src_v, dst_v, rows0, rows1, agg_sh, sem0, sem1):
    c = lax.axis_index("c")
    s = lax.axis_index("s")
    sl = pl.ds(s * SLAB, SLAB)
    pltpu.sync_copy(srcI.at[s], src_v)
    pltpu.sync_copy(dstI.at[s], dst_v)

    def quarter(tab, aout):
        pltpu.sync_copy(zrows, agg_sh.at[sl])
        plsc.subcore_barrier()
        pltpu.async_copy(tab.at[src_v.at[0]], rows0, sem0)

        def body(i, carry):
            j = 2 * i
            pltpu.async_copy(tab.at[src_v.at[j + 1]], rows1, sem1)
            pltpu.make_async_copy(tab.at[src_v.at[j]], rows0, sem0).wait()
            pltpu.sync_copy(rows0, agg_sh.at[dst_v.at[j]], add=True)

            @pl.when(j + 2 < CH)
            def _():
                pltpu.async_copy(tab.at[src_v.at[j + 2]], rows0, sem0)

            pltpu.make_async_copy(tab.at[src_v.at[j + 1]], rows1, sem1).wait()
            pltpu.sync_copy(rows1, agg_sh.at[dst_v.at[j + 1]], add=True)
            return carry

        lax.fori_loop(0, CH // 2, body, 0)
        plsc.subcore_barrier()
        pltpu.sync_copy(agg_sh.at[sl], aout.at[sl])

    @pl.when(c == 0)
    def _():
        quarter(t0, a0)
        quarter(t1, a1)

    @pl.when(c == 1)
    def _():
        quarter(t2, a2)
        quarter(t3, a3)


_agg_call = pl.kernel(
    _agg_body,
    out_type=tuple(jax.ShapeDtypeStruct((NPAD, DQ), jnp.float32)
                   for _ in range(4)),
    mesh=_sc_mesh,
    scratch_types=[
        pltpu.VMEM((CH, LCH), jnp.int32),
        pltpu.VMEM((CH, LCH), jnp.int32),
        pltpu.VMEM((LCH, DQ), jnp.float32),
        pltpu.VMEM((LCH, DQ), jnp.float32),
        pltpu.VMEM_SHARED((NPAD, DQ), jnp.float32),
        pltpu.SemaphoreType.DMA,
        pltpu.SemaphoreType.DMA,
    ],
    compiler_params=_sc_params,
)


def _store_quarters(hw, orefs):
    for q, oref in enumerate(orefs):
        oref[...] = hw[:, q * TQ:(q + 1) * TQ]


def _first_body(x_ref, dS_ref, w_ref, o0_ref, o1_ref, o2_ref, o3_ref):
    b = x_ref.shape[0]
    sn = lax.rsqrt(jnp.maximum(dS_ref[:, :1], 1.0)).reshape(b, 1, 1)
    hw = jnp.dot(
        x_ref[...].reshape(b * T, F), w_ref[...],
        preferred_element_type=jnp.float32,
    ).reshape(b, T, F) * sn
    _store_quarters(hw, (o0_ref, o1_ref, o2_ref, o3_ref))


def _mid_body(a0_ref, a1_ref, a2_ref, a3_ref, dD_ref, dS_ref, w_ref, b_ref,
              o0_ref, o1_ref, o2_ref, o3_ref):
    b = a0_ref.shape[0]
    agg = jnp.concatenate(
        [a0_ref[...], a1_ref[...], a2_ref[...], a3_ref[...]], axis=1)
    dn = lax.rsqrt(jnp.maximum(dD_ref[:, :1], 1.0)).reshape(b, 1, 1)
    h = jnp.maximum(agg * dn + b_ref[...].reshape(1, 1, F), 0.0)
    sn = lax.rsqrt(jnp.maximum(dS_ref[:, :1], 1.0)).reshape(b, 1, 1)
    hw = jnp.dot(
        h.reshape(b * T, F), w_ref[...],
        preferred_element_type=jnp.float32,
    ).reshape(b, T, F) * sn
    _store_quarters(hw, (o0_ref, o1_ref, o2_ref, o3_ref))


def _last_body(a0_ref, a1_ref, a2_ref, a3_ref, dD_ref, b_ref,
               wc0_ref, wc1_ref, wc2_ref, cb_ref, o_ref):
    b = a0_ref.shape[0]
    agg = jnp.concatenate(
        [a0_ref[...], a1_ref[...], a2_ref[...], a3_ref[...]], axis=1)
    dn = lax.rsqrt(jnp.maximum(dD_ref[:, :1], 1.0)).reshape(b, 1, 1)
    h = agg * dn + b_ref[...].reshape(1, 1, F)
    z = jnp.zeros((b, 1, F), jnp.float32)
    hp = jnp.concatenate([z, h, z], axis=1).reshape(b * (T + 2), F)

    def zmat(wc_ref):
        return jnp.dot(hp, wc_ref[...],
                       preferred_element_type=jnp.float32).reshape(b, T + 2, F)

    y = (zmat(wc0_ref)[:, 0:T] + zmat(wc1_ref)[:, 1:T + 1]
         + zmat(wc2_ref)[:, 2:T + 2]) + cb_ref[...].reshape(1, 1, F)
    o_ref[...] = y


def _node_spec(bn, t_len):
    return pl.BlockSpec((bn, t_len, F), lambda i: (i, 0, 0))


def _deg_spec(bn):
    return pl.BlockSpec((bn, 8), lambda i: (i, 0))


_w_spec = pl.BlockSpec((F, F), lambda i: (0, 0))
_b_spec = pl.BlockSpec((1, F), lambda i: (0, 0))


def _quarter_shapes(n):
    return tuple(jax.ShapeDtypeStruct((n, TQ, F), jnp.float32)
                 for _ in range(4))


def _first_call(x, degS, W):
    return pl.pallas_call(
        _first_body,
        grid=(GRID_N,),
        in_specs=[_node_spec(BN, T), _deg_spec(BN), _w_spec],
        out_specs=tuple(_node_spec(BN, TQ) for _ in range(4)),
        out_shape=_quarter_shapes(N),
    )(x, degS, W)


def _mid_call(a, degD, degS, W, bvec):
    return pl.pallas_call(
        _mid_body,
        grid=(GRID_P,),
        in_specs=[_node_spec(BNP, TQ)] * 4 + [_deg_spec(BNP), _deg_spec(BNP),
                                              _w_spec, _b_spec],
        out_specs=tuple(_node_spec(BNP, TQ) for _ in range(4)),
        out_shape=_quarter_shapes(NPAD),
    )(*a, degD, degS, W, bvec)


def _last_call(a, degD, bvec, wc0, wc1, wc2, cb):
    return pl.pallas_call(
        _last_body,
        grid=(GRID_P,),
        in_specs=[_node_spec(BNP, TQ)] * 4 + [_deg_spec(BNP), _b_spec,
                                              _w_spec, _w_spec, _w_spec,
                                              _b_spec],
        out_specs=_node_spec(BNP, T),
        out_shape=jax.ShapeDtypeStruct((NPAD, T, F), jnp.float32),
    )(*a, degD, bvec, wc0, wc1, wc2, cb)


@jax.jit
def kernel(temporal_features, edge_index, W1, b1, W2, b2, W3, b3, conv_w, conv_b):
    x = jnp.transpose(temporal_features, (0, 2, 1))
    src = edge_index[0]
    dst = edge_index[1]
    pad = EP - E
    srcA = jnp.concatenate([src, jnp.zeros((pad,), jnp.int32)]).reshape(NS, CH, LCH)
    srcD = jnp.concatenate([src, jnp.full((pad,), N, jnp.int32)]).reshape(NS, CH, LCH)
    dstI = jnp.concatenate([dst, jnp.full((pad,), N, jnp.int32)]).reshape(NS, CH, LCH)

    ones8 = jnp.ones((LCH, 8), jnp.float32)
    z8 = jnp.zeros((SLAB, 8), jnp.float32)
    zrows = jnp.zeros((SLAB, DQ), jnp.float32)

    degS, degD = _deg_call(srcD, dstI, ones8, z8)

    def agg(ts):
        a = _agg_call(*(t.reshape(-1, DQ) for t in ts), srcA, dstI, zrows)
        return tuple(q.reshape(NPAD, TQ, F) for q in a)

    t = _first_call(x, degS[:N], W1)
    a = agg(t)
    t = _mid_call(a, degD, degS, W2, b1.reshape(1, F))
    a = agg(t)
    t = _mid_call(a, degD, degS, W3, b2.reshape(1, F))
    a = agg(t)

    wc0 = conv_w[:, :, 0].T
    wc1 = conv_w[:, :, 1].T
    wc2 = conv_w[:, :, 2].T
    y = _last_call(a, degD, b3.reshape(1, F), wc0, wc1, wc2,
                   conv_b.reshape(1, F))
    return jnp.transpose(y[:N], (0, 2, 1))


# speedup vs baseline: 37.7316x; 1.4923x over previous
import jax
import jax.numpy as jnp
from jax import lax
from jax.experimental import pallas as pl
from jax.experimental.pallas import tpu as pltpu
from jax.experimental.pallas import tpu_sc as plsc

N = 10000
E = 160000
T = 12
F = 32
D = T * F
DQ = D // 4

NS = 16
CH = 80
LCH = 128
NBUF = 3
EP = NS * CH * LCH
NPAD = 10496
SLAB = NPAD // NS
BN = 400
BNP = 656
GRID_N = N // BN
GRID_P = NPAD // BNP

_sc_mesh = plsc.VectorSubcoreMesh(core_axis_name="c", subcore_axis_name="s")
_sc_params = pltpu.CompilerParams(use_tc_tiling_on_sc=False)


def _deg_body(srcI, dstI, ones_h, z8, degS, degD, idx_v, ones_v, deg_sh):
    c = lax.axis_index("c")
    s = lax.axis_index("s")
    sl = pl.ds(s * SLAB, SLAB)
    pltpu.sync_copy(z8, deg_sh.at[sl])
    pltpu.sync_copy(ones_h, ones_v)

    @pl.when(c == 0)
    def _():
        pltpu.sync_copy(srcI.at[s], idx_v)

    @pl.when(c == 1)
    def _():
        pltpu.sync_copy(dstI.at[s], idx_v)

    plsc.subcore_barrier()

    def body(j, carry):
        pltpu.sync_copy(ones_v, deg_sh.at[idx_v.at[j]], add=True)
        return carry

    lax.fori_loop(0, CH, body, 0)
    plsc.subcore_barrier()

    @pl.when(c == 0)
    def _():
        pltpu.sync_copy(deg_sh.at[sl], degS.at[sl])

    @pl.when(c == 1)
    def _():
        pltpu.sync_copy(deg_sh.at[sl], degD.at[sl])


_deg_call = pl.kernel(
    _deg_body,
    out_type=(
        jax.ShapeDtypeStruct((NPAD, 8), jnp.float32),
        jax.ShapeDtypeStruct((NPAD, 8), jnp.float32),
    ),
    mesh=_sc_mesh,
    scratch_types=[
        pltpu.VMEM((CH, LCH), jnp.int32),
        pltpu.VMEM((LCH, 8), jnp.float32),
        pltpu.VMEM_SHARED((NPAD, 8), jnp.float32),
    ],
    compiler_params=_sc_params,
)


def _agg_body(t0, t1, t2, t3, srcI, dstI, zrows, a0, a1, a2, a3,
              src_v, dst_v, rows, agg_sh, gsems, ssems):
    c = lax.axis_index("c")
    s = lax.axis_index("s")
    sl = pl.ds(s * SLAB, SLAB)
    pltpu.sync_copy(srcI.at[s], src_v)
    pltpu.sync_copy(dstI.at[s], dst_v)

    def quarter(tab, aout):
        pltpu.sync_copy(zrows, agg_sh.at[sl])
        plsc.subcore_barrier()
        for b in range(NBUF):
            pltpu.async_copy(tab.at[src_v.at[b]], rows.at[b], gsems.at[b])

        def body(i, carry):
            for b in range(NBUF):
                j = NBUF * i + b

                @pl.when(j < CH)
                def _():
                    pltpu.make_async_copy(
                        tab.at[src_v.at[j]], rows.at[b], gsems.at[b]).wait()
                    pltpu.async_copy(rows.at[b], agg_sh.at[dst_v.at[j]],
                                     ssems.at[b], add=True)

                    @pl.when(j + NBUF < CH)
                    def _():
                        pltpu.make_async_copy(
                            rows.at[b], agg_sh.at[dst_v.at[j]],
                            ssems.at[b]).wait()
                        pltpu.async_copy(tab.at[src_v.at[j + NBUF]],
                                         rows.at[b], gsems.at[b])

            return carry

        lax.fori_loop(0, (CH + NBUF - 1) // NBUF, body, 0)
        for b in range(NBUF):
            pltpu.make_async_copy(
                rows.at[b], agg_sh.at[dst_v.at[CH - NBUF + b]],
                ssems.at[b]).wait()
        plsc.subcore_barrier()
        pltpu.sync_copy(agg_sh.at[sl], aout.at[sl])

    @pl.when(c == 0)
    def _():
        quarter(t0, a0)
        quarter(t1, a1)

    @pl.when(c == 1)
    def _():
        quarter(t2, a2)
        quarter(t3, a3)


_agg_call = pl.kernel(
    _agg_body,
    out_type=tuple(jax.ShapeDtypeStruct((NPAD, DQ), jnp.float32)
                   for _ in range(4)),
    mesh=_sc_mesh,
    scratch_types=[
        pltpu.VMEM((CH, LCH), jnp.int32),
        pltpu.VMEM((CH, LCH), jnp.int32),
        pltpu.VMEM((NBUF, LCH, DQ), jnp.float32),
        pltpu.VMEM_SHARED((NPAD, DQ), jnp.float32),
        pltpu.SemaphoreType.DMA((NBUF,)),
        pltpu.SemaphoreType.DMA((NBUF,)),
    ],
    compiler_params=_sc_params,
)


def _store_quarters(hw, orefs):
    for q, oref in enumerate(orefs):
        oref[...] = hw[:, q * DQ:(q + 1) * DQ]


def _first_body(x_ref, dS_ref, w_ref, o0_ref, o1_ref, o2_ref, o3_ref):
    sn = lax.rsqrt(jnp.maximum(dS_ref[:, :1], 1.0))
    hw = jnp.dot(x_ref[...], w_ref[...],
                 preferred_element_type=jnp.float32) * sn
    _store_quarters(hw, (o0_ref, o1_ref, o2_ref, o3_ref))


def _mid_body(a0_ref, a1_ref, a2_ref, a3_ref, dD_ref, dS_ref, w_ref, b_ref,
              o0_ref, o1_ref, o2_ref, o3_ref):
    agg = jnp.concatenate(
        [a0_ref[...], a1_ref[...], a2_ref[...], a3_ref[...]], axis=1)
    dn = lax.rsqrt(jnp.maximum(dD_ref[:, :1], 1.0))
    h = jnp.maximum(agg * dn + b_ref[...], 0.0)
    sn = lax.rsqrt(jnp.maximum(dS_ref[:, :1], 1.0))
    hw = jnp.dot(h, w_ref[...], preferred_element_type=jnp.float32) * sn
    _store_quarters(hw, (o0_ref, o1_ref, o2_ref, o3_ref))


def _last_body(a0_ref, a1_ref, a2_ref, a3_ref, dD_ref, b_ref, wt_ref, cb_ref,
               o_ref):
    agg = jnp.concatenate(
        [a0_ref[...], a1_ref[...], a2_ref[...], a3_ref[...]], axis=1)
    dn = lax.rsqrt(jnp.maximum(dD_ref[:, :1], 1.0))
    h = agg * dn + b_ref[...]
    o_ref[...] = jnp.dot(h, wt_ref[...],
                         preferred_element_type=jnp.float32) + cb_ref[...]


def _row_spec(bn, width):
    return pl.BlockSpec((bn, width), lambda i: (i, 0))


_wbig_spec = pl.BlockSpec((D, D), lambda i: (0, 0))
_b_spec = pl.BlockSpec((1, D), lambda i: (0, 0))


def _quarter_shapes(n):
    return tuple(jax.ShapeDtypeStruct((n, DQ), jnp.float32) for _ in range(4))


def _first_call(x, degS, Wbig):
    return pl.pallas_call(
        _first_body,
        grid=(GRID_N,),
        in_specs=[_row_spec(BN, D), _row_spec(BN, 8), _wbig_spec],
        out_specs=tuple(_row_spec(BN, DQ) for _ in range(4)),
        out_shape=_quarter_shapes(N),
    )(x, degS, Wbig)


def _mid_call(a, degD, degS, Wbig, bvec):
    return pl.pallas_call(
        _mid_body,
        grid=(GRID_P,),
        in_specs=[_row_spec(BNP, DQ)] * 4 + [_row_spec(BNP, 8),
                                             _row_spec(BNP, 8),
                                             _wbig_spec, _b_spec],
        out_specs=tuple(_row_spec(BNP, DQ) for _ in range(4)),
        out_shape=_quarter_shapes(NPAD),
    )(*a, degD, degS, Wbig, bvec)


def _last_call(a, degD, bvec, WT, cb):
    return pl.pallas_call(
        _last_body,
        grid=(GRID_P,),
        in_specs=[_row_spec(BNP, DQ)] * 4 + [_row_spec(BNP, 8), _b_spec,
                                             _wbig_spec, _b_spec],
        out_specs=_row_spec(BNP, D),
        out_shape=jax.ShapeDtypeStruct((NPAD, D), jnp.float32),
    )(*a, degD, bvec, WT, cb)


@jax.jit
def kernel(temporal_features, edge_index, W1, b1, W2, b2, W3, b3, conv_w, conv_b):
    x = jnp.transpose(temporal_features, (0, 2, 1)).reshape(N, D)
    src = edge_index[0]
    dst = edge_index[1]
    pad = EP - E
    srcA = jnp.concatenate([src, jnp.zeros((pad,), jnp.int32)]).reshape(NS, CH, LCH)
    srcD = jnp.concatenate([src, jnp.full((pad,), N, jnp.int32)]).reshape(NS, CH, LCH)
    dstI = jnp.concatenate([dst, jnp.full((pad,), N, jnp.int32)]).reshape(NS, CH, LCH)

    ones8 = jnp.ones((LCH, 8), jnp.float32)
    z8 = jnp.zeros((SLAB, 8), jnp.float32)
    zrows = jnp.zeros((SLAB, DQ), jnp.float32)

    eyeT = jnp.eye(T, dtype=jnp.float32)
    W1big = jnp.kron(eyeT, W1)
    W2big = jnp.kron(eyeT, W2)
    W3big = jnp.kron(eyeT, W3)
    WT = sum(jnp.kron(jnp.eye(T, k=1 - k, dtype=jnp.float32),
                      conv_w[:, :, k].T) for k in range(3))
    b1t = jnp.tile(b1, T).reshape(1, D)
    b2t = jnp.tile(b2, T).reshape(1, D)
    b3t = jnp.tile(b3, T).reshape(1, D)
    cbt = jnp.tile(conv_b, T).reshape(1, D)

    degS, degD = _deg_call(srcD, dstI, ones8, z8)

    def agg(ts):
        a = _agg_call(*ts, srcA, dstI, zrows)
        return a

    t = _first_call(x, degS[:N], W1big)
    a = agg(t)
    t = _mid_call(a, degD, degS, W2big, b1t)
    a = agg(t)
    t = _mid_call(a, degD, degS, W3big, b2t)
    a = agg(t)

    y = _last_call(a, degD, b3t, WT, cbt)
    return jnp.transpose(y[:N].reshape(N, T, F), (0, 2, 1))


# speedup vs baseline: 39.6146x; 1.0499x over previous
import jax
import jax.numpy as jnp
from jax import lax
from jax.experimental import pallas as pl
from jax.experimental.pallas import tpu as pltpu
from jax.experimental.pallas import tpu_sc as plsc

N = 10000
E = 160000
T = 12
F = 32
D = T * F
DQ = D // 4

NS = 16
CH = 40
LCH = 256
NBUF = 2
EP = NS * CH * LCH
NPAD = 10112
SLAB = NPAD // NS
BN = 400
BNP = 632
GRID_N = N // BN
GRID_P = NPAD // BNP

_sc_mesh = plsc.VectorSubcoreMesh(core_axis_name="c", subcore_axis_name="s")
_sc_params = pltpu.CompilerParams(use_tc_tiling_on_sc=False)


def _deg_body(srcI, dstI, ones_h, z8, degS, degD, idx_v, ones_v, deg_sh):
    c = lax.axis_index("c")
    s = lax.axis_index("s")
    sl = pl.ds(s * SLAB, SLAB)
    pltpu.sync_copy(z8, deg_sh.at[sl])
    pltpu.sync_copy(ones_h, ones_v)

    @pl.when(c == 0)
    def _():
        pltpu.sync_copy(srcI.at[s], idx_v)

    @pl.when(c == 1)
    def _():
        pltpu.sync_copy(dstI.at[s], idx_v)

    plsc.subcore_barrier()

    def body(j, carry):
        pltpu.sync_copy(ones_v, deg_sh.at[idx_v.at[j]], add=True)
        return carry

    lax.fori_loop(0, CH, body, 0)
    plsc.subcore_barrier()

    @pl.when(c == 0)
    def _():
        pltpu.sync_copy(deg_sh.at[sl], degS.at[sl])

    @pl.when(c == 1)
    def _():
        pltpu.sync_copy(deg_sh.at[sl], degD.at[sl])


_deg_call = pl.kernel(
    _deg_body,
    out_type=(
        jax.ShapeDtypeStruct((NPAD, 8), jnp.float32),
        jax.ShapeDtypeStruct((NPAD, 8), jnp.float32),
    ),
    mesh=_sc_mesh,
    scratch_types=[
        pltpu.VMEM((CH, LCH), jnp.int32),
        pltpu.VMEM((LCH, 8), jnp.float32),
        pltpu.VMEM_SHARED((NPAD, 8), jnp.float32),
    ],
    compiler_params=_sc_params,
)


def _agg_body(t0, t1, t2, t3, srcI, dstI, zrows, a0, a1, a2, a3,
              src_v, dst_v, rows, agg_sh, gsems, ssems):
    c = lax.axis_index("c")
    s = lax.axis_index("s")
    sl = pl.ds(s * SLAB, SLAB)
    pltpu.sync_copy(srcI.at[s], src_v)
    pltpu.sync_copy(dstI.at[s], dst_v)

    def quarter(tab, aout):
        pltpu.sync_copy(zrows, agg_sh.at[sl])
        plsc.subcore_barrier()
        for b in range(NBUF):
            pltpu.async_copy(tab.at[src_v.at[b]], rows.at[b], gsems.at[b])

        def body(i, carry):
            for b in range(NBUF):
                j = NBUF * i + b

                @pl.when(j < CH)
                def _():
                    pltpu.make_async_copy(
                        tab.at[src_v.at[j]], rows.at[b], gsems.at[b]).wait()
                    pltpu.async_copy(rows.at[b], agg_sh.at[dst_v.at[j]],
                                     ssems.at[b], add=True)

                    @pl.when(j + NBUF < CH)
                    def _():
                        pltpu.make_async_copy(
                            rows.at[b], agg_sh.at[dst_v.at[j]],
                            ssems.at[b]).wait()
                        pltpu.async_copy(tab.at[src_v.at[j + NBUF]],
                                         rows.at[b], gsems.at[b])

            return carry

        lax.fori_loop(0, (CH + NBUF - 1) // NBUF, body, 0)
        for b in range(NBUF):
            pltpu.make_async_copy(
                rows.at[b], agg_sh.at[dst_v.at[CH - NBUF + b]],
                ssems.at[b]).wait()
        plsc.subcore_barrier()
        pltpu.sync_copy(agg_sh.at[sl], aout.at[sl])

    @pl.when(c == 0)
    def _():
        quarter(t0, a0)
        quarter(t1, a1)

    @pl.when(c == 1)
    def _():
        quarter(t2, a2)
        quarter(t3, a3)


_agg_call = pl.kernel(
    _agg_body,
    out_type=tuple(jax.ShapeDtypeStruct((NPAD, DQ), jnp.float32)
                   for _ in range(4)),
    mesh=_sc_mesh,
    scratch_types=[
        pltpu.VMEM((CH, LCH), jnp.int32),
        pltpu.VMEM((CH, LCH), jnp.int32),
        pltpu.VMEM((NBUF, LCH, DQ), jnp.float32),
        pltpu.VMEM_SHARED((NPAD, DQ), jnp.float32),
        pltpu.SemaphoreType.DMA((NBUF,)),
        pltpu.SemaphoreType.DMA((NBUF,)),
    ],
    compiler_params=_sc_params,
)


def _store_quarters(hw, orefs):
    for q, oref in enumerate(orefs):
        oref[...] = hw[:, q * DQ:(q + 1) * DQ]


def _first_body(x_ref, dS_ref, w_ref, o0_ref, o1_ref, o2_ref, o3_ref):
    sn = lax.rsqrt(jnp.maximum(dS_ref[:, :1], 1.0))
    hw = jnp.dot(x_ref[...], w_ref[...],
                 preferred_element_type=jnp.float32) * sn
    _store_quarters(hw, (o0_ref, o1_ref, o2_ref, o3_ref))


def _mid_body(a0_ref, a1_ref, a2_ref, a3_ref, dD_ref, dS_ref, w_ref, b_ref,
              o0_ref, o1_ref, o2_ref, o3_ref):
    agg = jnp.concatenate(
        [a0_ref[...], a1_ref[...], a2_ref[...], a3_ref[...]], axis=1)
    dn = lax.rsqrt(jnp.maximum(dD_ref[:, :1], 1.0))
    h = jnp.maximum(agg * dn + b_ref[...], 0.0)
    sn = lax.rsqrt(jnp.maximum(dS_ref[:, :1], 1.0))
    hw = jnp.dot(h, w_ref[...], preferred_element_type=jnp.float32) * sn
    _store_quarters(hw, (o0_ref, o1_ref, o2_ref, o3_ref))


def _last_body(a0_ref, a1_ref, a2_ref, a3_ref, dD_ref, b_ref, wt_ref, cb_ref,
               o_ref):
    agg = jnp.concatenate(
        [a0_ref[...], a1_ref[...], a2_ref[...], a3_ref[...]], axis=1)
    dn = lax.rsqrt(jnp.maximum(dD_ref[:, :1], 1.0))
    h = agg * dn + b_ref[...]
    o_ref[...] = jnp.dot(h, wt_ref[...],
                         preferred_element_type=jnp.float32) + cb_ref[...]


def _row_spec(bn, width):
    return pl.BlockSpec((bn, width), lambda i: (i, 0))


_wbig_spec = pl.BlockSpec((D, D), lambda i: (0, 0))
_b_spec = pl.BlockSpec((1, D), lambda i: (0, 0))


def _quarter_shapes(n):
    return tuple(jax.ShapeDtypeStruct((n, DQ), jnp.float32) for _ in range(4))


def _first_call(x, degS, Wbig):
    return pl.pallas_call(
        _first_body,
        grid=(GRID_N,),
        in_specs=[_row_spec(BN, D), _row_spec(BN, 8), _wbig_spec],
        out_specs=tuple(_row_spec(BN, DQ) for _ in range(4)),
        out_shape=_quarter_shapes(N),
    )(x, degS, Wbig)


def _mid_call(a, degD, degS, Wbig, bvec):
    return pl.pallas_call(
        _mid_body,
        grid=(GRID_P,),
        in_specs=[_row_spec(BNP, DQ)] * 4 + [_row_spec(BNP, 8),
                                             _row_spec(BNP, 8),
                                             _wbig_spec, _b_spec],
        out_specs=tuple(_row_spec(BNP, DQ) for _ in range(4)),
        out_shape=_quarter_shapes(NPAD),
    )(*a, degD, degS, Wbig, bvec)


def _last_call(a, degD, bvec, WT, cb):
    return pl.pallas_call(
        _last_body,
        grid=(GRID_P,),
        in_specs=[_row_spec(BNP, DQ)] * 4 + [_row_spec(BNP, 8), _b_spec,
                                             _wbig_spec, _b_spec],
        out_specs=_row_spec(BNP, D),
        out_shape=jax.ShapeDtypeStruct((NPAD, D), jnp.float32),
    )(*a, degD, bvec, WT, cb)


@jax.jit
def kernel(temporal_features, edge_index, W1, b1, W2, b2, W3, b3, conv_w, conv_b):
    x = jnp.transpose(temporal_features, (0, 2, 1)).reshape(N, D)
    src = edge_index[0]
    dst = edge_index[1]
    pad = EP - E
    srcA = jnp.concatenate([src, jnp.zeros((pad,), jnp.int32)]).reshape(NS, CH, LCH)
    srcD = jnp.concatenate([src, jnp.full((pad,), N, jnp.int32)]).reshape(NS, CH, LCH)
    dstI = jnp.concatenate([dst, jnp.full((pad,), N, jnp.int32)]).reshape(NS, CH, LCH)

    ones8 = jnp.ones((LCH, 8), jnp.float32)
    z8 = jnp.zeros((SLAB, 8), jnp.float32)
    zrows = jnp.zeros((SLAB, DQ), jnp.float32)

    eyeT = jnp.eye(T, dtype=jnp.float32)
    W1big = jnp.kron(eyeT, W1)
    W2big = jnp.kron(eyeT, W2)
    W3big = jnp.kron(eyeT, W3)
    WT = sum(jnp.kron(jnp.eye(T, k=1 - k, dtype=jnp.float32),
                      conv_w[:, :, k].T) for k in range(3))
    b1t = jnp.tile(b1, T).reshape(1, D)
    b2t = jnp.tile(b2, T).reshape(1, D)
    b3t = jnp.tile(b3, T).reshape(1, D)
    cbt = jnp.tile(conv_b, T).reshape(1, D)

    degS, degD = _deg_call(srcD, dstI, ones8, z8)

    def agg(ts):
        a = _agg_call(*ts, srcA, dstI, zrows)
        return a

    t = _first_call(x, degS[:N], W1big)
    a = agg(t)
    t = _mid_call(a, degD, degS, W2big, b1t)
    a = agg(t)
    t = _mid_call(a, degD, degS, W3big, b2t)
    a = agg(t)

    y = _last_call(a, degD, b3t, WT, cbt)
    return jnp.transpose(y[:N].reshape(N, T, F), (0, 2, 1))
